# Initial kernel scaffold; baseline (speedup 1.0000x reference)
import jax
import jax.numpy as jnp
from jax.experimental import pallas as pl


def kernel(rgb, depth):
    raise NotImplementedError("write your pallas kernel here")


# speedup vs baseline: 8.4399x; 8.4399x over previous
import functools

import jax
import jax.numpy as jnp
from jax import lax
from jax.experimental import pallas as pl
from jax.experimental.pallas import tpu as pltpu, tpu_sc as plsc

B = 4
H = 224
W = 224
HW = H * W
VOX = 64
ROWS_PER_CHUNK = 8
CHW = ROWS_PER_CHUNK * W
NUM_CHUNKS = H // ROWS_PER_CHUNK
VREGS_PER_ROW = W // 16
SLAB_X = 8
SLAB = SLAB_X * VOX * VOX
SENT = 1 << 30


def _voxel_body(rgb_hbm, depth_hbm, out_hbm,
                slab_r, slab_g, slab_b, dbuf, rbuf, gbuf, bbuf, sbuf):
    wid = lax.axis_index("c") * 16 + lax.axis_index("s")
    bb = wid // 8
    ss = wid % 8

    iota_i = lax.iota(jnp.int32, 16)
    iota_f = iota_i.astype(jnp.float32)
    zeros16 = jnp.zeros((16,), jnp.float32)

    sbuf[pl.ds(16, 16)] = jnp.full((16,), jnp.int32(0x7FFFFFF0)) | iota_i

    def _zero(i, carry):
        slab_r[pl.ds(i * 16, 16)] = zeros16
        slab_g[pl.ds(i * 16, 16)] = zeros16
        slab_b[pl.ds(i * 16, 16)] = zeros16
        return carry

    lax.fori_loop(0, SLAB // 16, _zero, 0)

    def _chunk(ck, carry):
        off_d = bb * HW + ck * CHW
        pltpu.sync_copy(depth_hbm.at[pl.ds(off_d, CHW)], dbuf)
        off_rgb = (bb * 3) * HW + ck * CHW
        pltpu.sync_copy(rgb_hbm.at[pl.ds(off_rgb, CHW)], rbuf)
        pltpu.sync_copy(rgb_hbm.at[pl.ds(off_rgb + HW, CHW)], gbuf)
        pltpu.sync_copy(rgb_hbm.at[pl.ds(off_rgb + 2 * HW, CHW)], bbuf)

        def _row(r, carry2):
            row = ck * ROWS_PER_CHUNK + r
            vminus = (row.astype(jnp.float32) - 112.0)
            for j in range(VREGS_PER_ROW):
                base = r * W + j * 16
                dv = dbuf[pl.ds(base, 16)]
                u_f = iota_f + float(j * 16)
                xf = (u_f - 112.0) * dv / 112.0
                yf = vminus * dv / 112.0
                ix = ((xf + 1.0) / 2.0 * 64.0).astype(jnp.int32)
                iy = ((yf + 1.0) / 2.0 * 64.0).astype(jnp.int32)
                iz = ((dv + 1.0) / 2.0 * 64.0).astype(jnp.int32)
                ok = (dv > 0.1) & (dv < 10.0)
                ok &= (ix >= 0) & (ix < 64) & (iy >= 0) & (iy < 64)
                ok &= (iz >= 0) & (iz < 64)
                ok &= (ix >> 3) == ss
                lidx = ((ix & 7) << 12) | (iy << 6) | iz
                key = jnp.where(ok, (lidx << 4) | iota_i, SENT | iota_i)
                sk = lax.sort(key, dimension=0, is_stable=False)
                sbuf[pl.ds(0, 16)] = sk
                skn = sbuf[pl.ds(1, 16)]
                m = ((sk >> 4) != (skn >> 4)) & (sk < SENT)
                lane = sk & 15
                _, mo = plsc.sort_key_val(lane, m.astype(jnp.int32))
                mask = mo == 1
                plsc.store_scatter(slab_r, [lidx], rbuf[pl.ds(base, 16)],
                                   mask=mask)
                plsc.store_scatter(slab_g, [lidx], gbuf[pl.ds(base, 16)],
                                   mask=mask)
                plsc.store_scatter(slab_b, [lidx], bbuf[pl.ds(base, 16)],
                                   mask=mask)
            return carry2

        lax.fori_loop(0, ROWS_PER_CHUNK, _row, 0)
        return carry

    lax.fori_loop(0, NUM_CHUNKS, _chunk, 0)

    out_base = ((bb * 3) * 8 + ss) * SLAB
    pltpu.sync_copy(slab_r, out_hbm.at[pl.ds(out_base, SLAB)])
    pltpu.sync_copy(slab_g, out_hbm.at[pl.ds(out_base + 8 * SLAB, SLAB)])
    pltpu.sync_copy(slab_b, out_hbm.at[pl.ds(out_base + 16 * SLAB, SLAB)])


@jax.jit
def kernel(rgb, depth):
    rgb_flat = rgb.reshape(B * 3 * HW)
    depth_flat = depth.reshape(B * HW)
    mesh = plsc.VectorSubcoreMesh(core_axis_name="c", subcore_axis_name="s")
    call = functools.partial(
        pl.kernel,
        mesh=mesh,
        compiler_params=pltpu.CompilerParams(needs_layout_passes=False),
        out_type=jax.ShapeDtypeStruct((B * 3 * 8 * SLAB,), jnp.float32),
        scratch_types=[
            pltpu.VMEM((SLAB,), jnp.float32),
            pltpu.VMEM((SLAB,), jnp.float32),
            pltpu.VMEM((SLAB,), jnp.float32),
            pltpu.VMEM((CHW,), jnp.float32),
            pltpu.VMEM((CHW,), jnp.float32),
            pltpu.VMEM((CHW,), jnp.float32),
            pltpu.VMEM((CHW,), jnp.float32),
            pltpu.VMEM((32,), jnp.int32),
        ],
    )(_voxel_body)
    out_flat = call(rgb_flat, depth_flat)
    return out_flat.reshape(B, 3, VOX, VOX, VOX)


# speedup vs baseline: 27.4435x; 3.2516x over previous
import functools

import jax
import jax.numpy as jnp
from jax import lax
from jax.experimental import pallas as pl
from jax.experimental.pallas import tpu as pltpu, tpu_sc as plsc

B = 4
H = 224
W = 224
HW = H * W
VOX = 64
ROWS_PER_CHUNK = 8
CHW = ROWS_PER_CHUNK * W
NUM_CHUNKS = H // ROWS_PER_CHUNK
VREGS_PER_ROW = W // 16
GROUP = 7
SLAB = 8 * VOX * VOX


def _voxel_body(rgb_hbm, depth_hbm, um_hbm, out_hbm,
                slab_r, slab_g, slab_b, dbuf, rbuf, gbuf, bbuf, ubuf,
                sem0, sem1, semo):
    wid = lax.axis_index("c") * 16 + lax.axis_index("s")
    bb = wid // 8
    ss = wid % 8

    pltpu.sync_copy(um_hbm, ubuf)

    zeros16 = jnp.zeros((16,), jnp.float32)

    def _zero(i, carry):
        slab_r[pl.ds(i * 16, 16)] = zeros16
        slab_g[pl.ds(i * 16, 16)] = zeros16
        slab_b[pl.ds(i * 16, 16)] = zeros16
        return carry

    lax.fori_loop(0, SLAB // 16, _zero, 0, unroll=4)

    def _in_copies(ck, slot, sem):
        off_d = bb * HW + ck * CHW
        off_rgb = (bb * 3) * HW + ck * CHW
        return (
            (depth_hbm.at[pl.ds(off_d, CHW)],
             dbuf.at[pl.ds(slot * CHW, CHW)], sem),
            (rgb_hbm.at[pl.ds(off_rgb, CHW)],
             rbuf.at[pl.ds(slot * CHW, CHW)], sem),
            (rgb_hbm.at[pl.ds(off_rgb + HW, CHW)],
             gbuf.at[pl.ds(slot * CHW, CHW)], sem),
            (rgb_hbm.at[pl.ds(off_rgb + 2 * HW, CHW)],
             bbuf.at[pl.ds(slot * CHW, CHW)], sem),
        )

    def _start(ck, slot, sem):
        for src, dst, s in _in_copies(ck, slot, sem):
            pltpu.async_copy(src, dst, s)

    def _wait(ck, slot, sem):
        for src, dst, s in _in_copies(ck, slot, sem):
            pltpu.make_async_copy(src, dst, s).wait()

    def _compute_chunk(ck, slot):
        def _row(r, carry2):
            row = ck * ROWS_PER_CHUNK + r
            vminus = row.astype(jnp.float32) - 112.0
            for g in range(VREGS_PER_ROW // GROUP):
                lidxs, wins, rvs, gvs, bvs = [], [], [], [], []
                for j in range(g * GROUP, (g + 1) * GROUP):
                    base = slot * CHW + r * W + j * 16
                    dv = dbuf[pl.ds(base, 16)]
                    u_m = ubuf[pl.ds(j * 16, 16)]
                    xf = u_m * dv / 112.0
                    yf = vminus * dv / 112.0
                    ix = ((xf + 1.0) / 2.0 * 64.0).astype(jnp.int32)
                    iy = ((yf + 1.0) / 2.0 * 64.0).astype(jnp.int32)
                    iz = ((dv + 1.0) / 2.0 * 64.0).astype(jnp.int32)
                    ok = (dv > 0.1) & (iy < 64) & (iz < 64)
                    ok &= (ix >> 3) == ss
                    lidx = ((ix & 7) << 12) | (iy << 6) | iz
                    _, win = plsc.scan_count(lidx, mask=ok)
                    lidxs.append(lidx)
                    wins.append(win)
                    rvs.append(rbuf[pl.ds(base, 16)])
                    gvs.append(gbuf[pl.ds(base, 16)])
                    bvs.append(bbuf[pl.ds(base, 16)])
                for t in range(GROUP):
                    plsc.store_scatter(slab_r, [lidxs[t]], rvs[t],
                                       mask=wins[t])
                    plsc.store_scatter(slab_g, [lidxs[t]], gvs[t],
                                       mask=wins[t])
                    plsc.store_scatter(slab_b, [lidxs[t]], bvs[t],
                                       mask=wins[t])
            return carry2

        lax.fori_loop(0, ROWS_PER_CHUNK, _row, 0)

    _start(0, 0, sem0)

    def _pair(k, carry):
        ck0 = 2 * k
        _start(ck0 + 1, 1, sem1)
        _wait(ck0, 0, sem0)
        _compute_chunk(ck0, 0)

        @pl.when(ck0 + 2 < NUM_CHUNKS)
        def _():
            _start(ck0 + 2, 0, sem0)

        _wait(ck0 + 1, 1, sem1)
        _compute_chunk(ck0 + 1, 1)
        return carry

    lax.fori_loop(0, NUM_CHUNKS // 2, _pair, 0)

    out_base = ((bb * 3) * 8 + ss) * SLAB
    out_copies = (
        (slab_r, out_hbm.at[pl.ds(out_base, SLAB)], semo),
        (slab_g, out_hbm.at[pl.ds(out_base + 8 * SLAB, SLAB)], semo),
        (slab_b, out_hbm.at[pl.ds(out_base + 16 * SLAB, SLAB)], semo),
    )
    for src, dst, s in out_copies:
        pltpu.async_copy(src, dst, s)
    for src, dst, s in out_copies:
        pltpu.make_async_copy(src, dst, s).wait()


@jax.jit
def kernel(rgb, depth):
    rgb_flat = rgb.reshape(B * 3 * HW)
    depth_flat = depth.reshape(B * HW)
    um = jnp.arange(W, dtype=jnp.float32) - 112.0
    mesh = plsc.VectorSubcoreMesh(core_axis_name="c", subcore_axis_name="s")
    call = functools.partial(
        pl.kernel,
        mesh=mesh,
        compiler_params=pltpu.CompilerParams(needs_layout_passes=False),
        out_type=jax.ShapeDtypeStruct((B * 3 * 8 * SLAB,), jnp.float32),
        scratch_types=[
            pltpu.VMEM((SLAB,), jnp.float32),
            pltpu.VMEM((SLAB,), jnp.float32),
            pltpu.VMEM((SLAB,), jnp.float32),
            pltpu.VMEM((2 * CHW,), jnp.float32),
            pltpu.VMEM((2 * CHW,), jnp.float32),
            pltpu.VMEM((2 * CHW,), jnp.float32),
            pltpu.VMEM((2 * CHW,), jnp.float32),
            pltpu.VMEM((W,), jnp.float32),
            pltpu.SemaphoreType.DMA,
            pltpu.SemaphoreType.DMA,
            pltpu.SemaphoreType.DMA,
        ],
    )(_voxel_body)
    out_flat = call(rgb_flat, depth_flat, um)
    return out_flat.reshape(B, 3, VOX, VOX, VOX)


# speedup vs baseline: 28.6222x; 1.0429x over previous
import functools

import jax
import jax.numpy as jnp
from jax import lax
from jax.experimental import pallas as pl
from jax.experimental.pallas import tpu as pltpu, tpu_sc as plsc

B = 4
H = 224
W = 224
HW = H * W
VOX = 64
ROWS_PER_CHUNK = 8
CHW = ROWS_PER_CHUNK * W
NUM_CHUNKS = H // ROWS_PER_CHUNK
VREGS_PER_ROW = W // 16
GROUP = 7
SLAB = 8 * VOX * VOX
ONE_MINUS = float(1.0 - 2.0 ** -24)


def _voxel_body(rgb_hbm, depth_hbm, um_hbm, out_hbm,
                slab_r, slab_g, slab_b, dbuf, rbuf, gbuf, bbuf, ubuf,
                sem0, sem1, semo):
    wid = lax.axis_index("c") * 16 + lax.axis_index("s")
    bb = wid // 8
    ss = wid % 8

    pltpu.sync_copy(um_hbm, ubuf)

    zeros16 = jnp.zeros((16,), jnp.float32)

    def _zero(i, carry):
        slab_r[pl.ds(i * 16, 16)] = zeros16
        slab_g[pl.ds(i * 16, 16)] = zeros16
        slab_b[pl.ds(i * 16, 16)] = zeros16
        return carry

    def _in_copies(ck, slot, sem):
        off_d = bb * HW + ck * CHW
        off_rgb = (bb * 3) * HW + ck * CHW
        return (
            (depth_hbm.at[pl.ds(off_d, CHW)],
             dbuf.at[pl.ds(slot * CHW, CHW)], sem),
            (rgb_hbm.at[pl.ds(off_rgb, CHW)],
             rbuf.at[pl.ds(slot * CHW, CHW)], sem),
            (rgb_hbm.at[pl.ds(off_rgb + HW, CHW)],
             gbuf.at[pl.ds(slot * CHW, CHW)], sem),
            (rgb_hbm.at[pl.ds(off_rgb + 2 * HW, CHW)],
             bbuf.at[pl.ds(slot * CHW, CHW)], sem),
        )

    def _start(ck, slot, sem):
        for src, dst, s in _in_copies(ck, slot, sem):
            pltpu.async_copy(src, dst, s)

    def _wait(ck, slot, sem):
        for src, dst, s in _in_copies(ck, slot, sem):
            pltpu.make_async_copy(src, dst, s).wait()

    def _compute_chunk(ck, slot):
        ums = [ubuf[pl.ds(j * 16, 16)] for j in range(VREGS_PER_ROW)]

        def _row(r, carry2):
            row = ck * ROWS_PER_CHUNK + r
            vminus = row.astype(jnp.float32) - 112.0
            for g in range(VREGS_PER_ROW // GROUP):
                lidxs, wins, rvs, gvs, bvs = [], [], [], [], []
                for j in range(g * GROUP, (g + 1) * GROUP):
                    base = slot * CHW + r * W + j * 16
                    dv = dbuf[pl.ds(base, 16)]
                    xf = ums[j] * dv / 112.0
                    yf = vminus * dv / 112.0
                    ix = ((xf + 1.0) * 32.0).astype(jnp.int32)
                    iy = ((yf + 1.0) * 32.0).astype(jnp.int32)
                    iz = ((dv + 1.0) * 32.0).astype(jnp.int32)
                    ok = (dv > 0.1) & (dv < ONE_MINUS) & (iy < 64)
                    ok &= (ix >> 3) == ss
                    lidx = ((ix & 7) << 12) | (iy << 6) | iz
                    _, win = plsc.scan_count(lidx, mask=ok)
                    lidxs.append(lidx)
                    wins.append(win)
                    rvs.append(rbuf[pl.ds(base, 16)])
                    gvs.append(gbuf[pl.ds(base, 16)])
                    bvs.append(bbuf[pl.ds(base, 16)])
                for t in range(GROUP):
                    plsc.store_scatter(slab_r, [lidxs[t]], rvs[t],
                                       mask=wins[t])
                    plsc.store_scatter(slab_g, [lidxs[t]], gvs[t],
                                       mask=wins[t])
                    plsc.store_scatter(slab_b, [lidxs[t]], bvs[t],
                                       mask=wins[t])
            return carry2

        lax.fori_loop(0, ROWS_PER_CHUNK, _row, 0)

    _start(0, 0, sem0)
    lax.fori_loop(0, SLAB // 16, _zero, 0, unroll=4)

    def _pair(k, carry):
        ck0 = 2 * k
        _start(ck0 + 1, 1, sem1)
        _wait(ck0, 0, sem0)
        _compute_chunk(ck0, 0)

        @pl.when(ck0 + 2 < NUM_CHUNKS)
        def _():
            _start(ck0 + 2, 0, sem0)

        _wait(ck0 + 1, 1, sem1)
        _compute_chunk(ck0 + 1, 1)
        return carry

    lax.fori_loop(0, NUM_CHUNKS // 2, _pair, 0)

    out_base = ((bb * 3) * 8 + ss) * SLAB
    out_copies = (
        (slab_r, out_hbm.at[pl.ds(out_base, SLAB)], semo),
        (slab_g, out_hbm.at[pl.ds(out_base + 8 * SLAB, SLAB)], semo),
        (slab_b, out_hbm.at[pl.ds(out_base + 16 * SLAB, SLAB)], semo),
    )
    for src, dst, s in out_copies:
        pltpu.async_copy(src, dst, s)
    for src, dst, s in out_copies:
        pltpu.make_async_copy(src, dst, s).wait()


@jax.jit
def kernel(rgb, depth):
    rgb_flat = rgb.reshape(B * 3 * HW)
    depth_flat = depth.reshape(B * HW)
    um = jnp.arange(W, dtype=jnp.float32) - 112.0
    mesh = plsc.VectorSubcoreMesh(core_axis_name="c", subcore_axis_name="s")
    call = functools.partial(
        pl.kernel,
        mesh=mesh,
        compiler_params=pltpu.CompilerParams(needs_layout_passes=False),
        out_type=jax.ShapeDtypeStruct((B * 3 * 8 * SLAB,), jnp.float32),
        scratch_types=[
            pltpu.VMEM((SLAB,), jnp.float32),
            pltpu.VMEM((SLAB,), jnp.float32),
            pltpu.VMEM((SLAB,), jnp.float32),
            pltpu.VMEM((2 * CHW,), jnp.float32),
            pltpu.VMEM((2 * CHW,), jnp.float32),
            pltpu.VMEM((2 * CHW,), jnp.float32),
            pltpu.VMEM((2 * CHW,), jnp.float32),
            pltpu.VMEM((W,), jnp.float32),
            pltpu.SemaphoreType.DMA,
            pltpu.SemaphoreType.DMA,
            pltpu.SemaphoreType.DMA,
        ],
    )(_voxel_body)
    out_flat = call(rgb_flat, depth_flat, um)
    return out_flat.reshape(B, 3, VOX, VOX, VOX)


# speedup vs baseline: 30.1398x; 1.0530x over previous
import functools

import jax
import jax.numpy as jnp
from jax import lax
from jax.experimental import pallas as pl
from jax.experimental.pallas import tpu as pltpu, tpu_sc as plsc

B = 4
H = 224
W = 224
VOX = 64
RPC = 8
NUM_CHUNKS = H // RPC
VREGS_PER_ROW = W // 16
GROUP = 7
SLAB = 8 * VOX * VOX
ONE_MINUS = float(1.0 - 2.0 ** -24)


def _voxel_body(rgb_hbm, depth_hbm, um_hbm, out_hbm,
                slab_r, slab_g, slab_b,
                da, db_, ra, rb, ga, gb, ba, bb_, ubuf,
                sem0, sem1, semo):
    wid = lax.axis_index("c") * 16 + lax.axis_index("s")
    bb = wid // 8
    ss = wid % 8

    pltpu.sync_copy(um_hbm, ubuf)

    zeros16 = jnp.zeros((16,), jnp.float32)

    def _zero(i, carry):
        slab_r[pl.ds(i * 16, 16)] = zeros16
        slab_g[pl.ds(i * 16, 16)] = zeros16
        slab_b[pl.ds(i * 16, 16)] = zeros16
        return carry

    def _in_copies(ck, slot, sem):
        r0 = ck * RPC
        cps = []
        for plane, bufa, bufb in (
            (depth_hbm.at[bb, 0], da, db_),
            (rgb_hbm.at[bb, 0], ra, rb),
            (rgb_hbm.at[bb, 1], ga, gb),
            (rgb_hbm.at[bb, 2], ba, bb_),
        ):
            cps.append((plane.at[pl.ds(r0, RPC), pl.ds(0, 128)],
                        bufa.at[slot], sem))
            cps.append((plane.at[pl.ds(r0, RPC), pl.ds(128, 96)],
                        bufb.at[slot], sem))
        return cps

    def _start(ck, slot, sem):
        for src, dst, s in _in_copies(ck, slot, sem):
            pltpu.async_copy(src, dst, s)

    def _wait(ck, slot, sem):
        for src, dst, s in _in_copies(ck, slot, sem):
            pltpu.make_async_copy(src, dst, s).wait()

    def _compute_chunk(ck, slot):
        ums = [ubuf[pl.ds(j * 16, 16)] for j in range(VREGS_PER_ROW)]

        def _row(r, carry2):
            row = ck * RPC + r
            vminus = row.astype(jnp.float32) - 112.0
            for g in range(VREGS_PER_ROW // GROUP):
                xys, wins, rvs, gvs, bvs = [], [], [], [], []
                for j in range(g * GROUP, (g + 1) * GROUP):
                    if j < 8:
                        sl = (slot, r, pl.ds(16 * j, 16))
                        dv = da[sl]
                    else:
                        sl = (slot, r, pl.ds(16 * j - 128, 16))
                        dv = db_[sl]
                    xf = ums[j] * dv / 112.0
                    yf = vminus * dv / 112.0
                    ix = ((xf + 1.0) * 32.0).astype(jnp.int32)
                    iy = ((yf + 1.0) * 32.0).astype(jnp.int32)
                    iz = ((dv + 1.0) * 32.0).astype(jnp.int32)
                    ok = (dv > 0.1) & (dv < ONE_MINUS) & (iy < 64)
                    ok &= (ix >> 3) == ss
                    lidx = ((ix & 7) << 12) | (iy << 6) | iz
                    _, win = plsc.scan_count(lidx, mask=ok)
                    xys.append(lidx)
                    wins.append(win)
                    rvs.append(ra[sl] if j < 8 else rb[sl])
                    gvs.append(ga[sl] if j < 8 else gb[sl])
                    bvs.append(ba[sl] if j < 8 else bb_[sl])
                for t in range(GROUP):
                    plsc.store_scatter(slab_r, [xys[t]], rvs[t],
                                       mask=wins[t])
                    plsc.store_scatter(slab_g, [xys[t]], gvs[t],
                                       mask=wins[t])
                    plsc.store_scatter(slab_b, [xys[t]], bvs[t],
                                       mask=wins[t])
            return carry2

        lax.fori_loop(0, RPC, _row, 0)

    _start(0, 0, sem0)
    lax.fori_loop(0, SLAB // 16, _zero, 0, unroll=4)

    def _pair(k, carry):
        ck0 = 2 * k
        _start(ck0 + 1, 1, sem1)
        _wait(ck0, 0, sem0)
        _compute_chunk(ck0, 0)

        @pl.when(ck0 + 2 < NUM_CHUNKS)
        def _():
            _start(ck0 + 2, 0, sem0)

        _wait(ck0 + 1, 1, sem1)
        _compute_chunk(ck0 + 1, 1)
        return carry

    lax.fori_loop(0, NUM_CHUNKS // 2, _pair, 0)

    out_base = ((bb * 3) * 8 + ss) * SLAB
    out_copies = (
        (slab_r, out_hbm.at[pl.ds(out_base, SLAB)], semo),
        (slab_g, out_hbm.at[pl.ds(out_base + 8 * SLAB, SLAB)], semo),
        (slab_b, out_hbm.at[pl.ds(out_base + 16 * SLAB, SLAB)], semo),
    )
    for src, dst, s in out_copies:
        pltpu.async_copy(src, dst, s)
    for src, dst, s in out_copies:
        pltpu.make_async_copy(src, dst, s).wait()


@jax.jit
def kernel(rgb, depth):
    um = jnp.arange(W, dtype=jnp.float32) - 112.0
    mesh = plsc.VectorSubcoreMesh(core_axis_name="c", subcore_axis_name="s")
    call = functools.partial(
        pl.kernel,
        mesh=mesh,
        compiler_params=pltpu.CompilerParams(needs_layout_passes=False),
        out_type=jax.ShapeDtypeStruct((B * 3 * 8 * SLAB,), jnp.float32),
        scratch_types=[
            pltpu.VMEM((SLAB,), jnp.float32),
            pltpu.VMEM((SLAB,), jnp.float32),
            pltpu.VMEM((SLAB,), jnp.float32),
            pltpu.VMEM((2, RPC, 128), jnp.float32),
            pltpu.VMEM((2, RPC, 96), jnp.float32),
            pltpu.VMEM((2, RPC, 128), jnp.float32),
            pltpu.VMEM((2, RPC, 96), jnp.float32),
            pltpu.VMEM((2, RPC, 128), jnp.float32),
            pltpu.VMEM((2, RPC, 96), jnp.float32),
            pltpu.VMEM((2, RPC, 128), jnp.float32),
            pltpu.VMEM((2, RPC, 96), jnp.float32),
            pltpu.VMEM((W,), jnp.float32),
            pltpu.SemaphoreType.DMA,
            pltpu.SemaphoreType.DMA,
            pltpu.SemaphoreType.DMA,
        ],
    )(_voxel_body)
    out_flat = call(rgb, depth, um)
    return out_flat.reshape(B, 3, VOX, VOX, VOX)


# speedup vs baseline: 35.4923x; 1.1776x over previous
import functools

import jax
import jax.numpy as jnp
from jax import lax
from jax.experimental import pallas as pl
from jax.experimental.pallas import tpu as pltpu, tpu_sc as plsc

B = 4
H = 224
W = 224
VOX = 64
RPC = 8
NUM_CHUNKS = H // RPC
VREGS_PER_ROW = W // 16
GROUP = 7
ZH = 32
SLAB = 8 * VOX * ZH
ONE_MINUS = float(1.0 - 2.0 ** -24)


def _voxel_body(rgb_hbm, depth_hbm, um_hbm, out_hbm,
                slab_r, slab_g, slab_b,
                da, db_, ra, rb, ga, gb, ba, bb_, ubuf, stg0, stg1,
                sem0, sem1, semo):
    wid = lax.axis_index("c") * 16 + lax.axis_index("s")
    bb = wid // 8
    ss = wid % 8

    pltpu.sync_copy(um_hbm, ubuf)

    zeros16 = jnp.zeros((16,), jnp.float32)

    def _zero(i, carry):
        slab_r[pl.ds(i * 16, 16)] = zeros16
        slab_g[pl.ds(i * 16, 16)] = zeros16
        slab_b[pl.ds(i * 16, 16)] = zeros16
        return carry

    def _in_copies(ck, slot, sem):
        r0 = ck * RPC
        cps = []
        for plane, bufa, bufb in (
            (depth_hbm.at[bb, 0], da, db_),
            (rgb_hbm.at[bb, 0], ra, rb),
            (rgb_hbm.at[bb, 1], ga, gb),
            (rgb_hbm.at[bb, 2], ba, bb_),
        ):
            cps.append((plane.at[pl.ds(r0, RPC), pl.ds(0, 128)],
                        bufa.at[slot], sem))
            cps.append((plane.at[pl.ds(r0, RPC), pl.ds(128, 96)],
                        bufb.at[slot], sem))
        return cps

    def _start(ck, slot, sem):
        for src, dst, s in _in_copies(ck, slot, sem):
            pltpu.async_copy(src, dst, s)

    def _wait(ck, slot, sem):
        for src, dst, s in _in_copies(ck, slot, sem):
            pltpu.make_async_copy(src, dst, s).wait()

    def _compute_chunk(ck, slot):
        ums = [ubuf[pl.ds(j * 16, 16)] for j in range(VREGS_PER_ROW)]

        def _row(r, carry2):
            row = ck * RPC + r
            vminus = row.astype(jnp.float32) - 112.0
            for g in range(VREGS_PER_ROW // GROUP):
                xys, wins, rvs, gvs, bvs = [], [], [], [], []
                for j in range(g * GROUP, (g + 1) * GROUP):
                    if j < 8:
                        sl = (slot, r, pl.ds(16 * j, 16))
                        dv = da[sl]
                    else:
                        sl = (slot, r, pl.ds(16 * j - 128, 16))
                        dv = db_[sl]
                    xf = ums[j] * dv / 112.0
                    yf = vminus * dv / 112.0
                    ix = ((xf + 1.0) * 32.0).astype(jnp.int32)
                    iy = ((yf + 1.0) * 32.0).astype(jnp.int32)
                    iz = ((dv + 1.0) * 32.0).astype(jnp.int32)
                    ok = (dv > 0.1) & (dv < ONE_MINUS) & (iy < 64)
                    ok &= (ix >> 3) == ss
                    lidx = ((ix & 7) << 11) | (iy << 5) | (iz & 31)
                    _, win = plsc.scan_count(lidx, mask=ok)
                    xys.append(lidx)
                    wins.append(win)
                    rvs.append(ra[sl] if j < 8 else rb[sl])
                    gvs.append(ga[sl] if j < 8 else gb[sl])
                    bvs.append(ba[sl] if j < 8 else bb_[sl])
                for t in range(GROUP):
                    plsc.store_scatter(slab_r, [xys[t]], rvs[t],
                                       mask=wins[t])
                    plsc.store_scatter(slab_g, [xys[t]], gvs[t],
                                       mask=wins[t])
                    plsc.store_scatter(slab_b, [xys[t]], bvs[t],
                                       mask=wins[t])
            return carry2

        lax.fori_loop(0, RPC, _row, 0)

    _start(0, 0, sem0)
    lax.fori_loop(0, SLAB // 16, _zero, 0, unroll=4)

    def _pair(k, carry):
        ck0 = 2 * k
        _start(ck0 + 1, 1, sem1)
        _wait(ck0, 0, sem0)
        _compute_chunk(ck0, 0)

        @pl.when(ck0 + 2 < NUM_CHUNKS)
        def _():
            _start(ck0 + 2, 0, sem0)

        _wait(ck0 + 1, 1, sem1)
        _compute_chunk(ck0 + 1, 1)
        return carry

    lax.fori_loop(0, NUM_CHUNKS // 2, _pair, 0)

    zeros_row = jnp.zeros((16,), jnp.float32)

    def _zstage(y, carry):
        stg0[y, pl.ds(0, 16)] = zeros_row
        stg0[y, pl.ds(16, 16)] = zeros_row
        stg1[y, pl.ds(0, 16)] = zeros_row
        stg1[y, pl.ds(16, 16)] = zeros_row
        return carry

    lax.fori_loop(0, VOX, _zstage, 0, unroll=4)

    planes = [(c, xl) for c in range(3) for xl in range(8)]
    slabs = (slab_r, slab_g, slab_b)

    def _repack(slab, stg, xl):
        def _rrow(y, carry):
            base = (xl * VOX + y) * ZH
            stg[y, pl.ds(32, 16)] = slab[pl.ds(base, 16)]
            stg[y, pl.ds(48, 16)] = slab[pl.ds(base + 16, 16)]
            return carry

        lax.fori_loop(0, VOX, _rrow, 0, unroll=4)

    for i, (c, xl) in enumerate(planes):
        stg = (stg0, stg1)[i % 2]
        sem = (sem0, sem1)[i % 2]
        if i >= 2:
            pc, pxl = planes[i - 2]
            pltpu.make_async_copy(
                stg,
                out_hbm.at[bb, pc, 8 * ss + pxl], sem).wait()
        _repack(slabs[c], stg, xl)
        pltpu.async_copy(stg,
                         out_hbm.at[bb, c, 8 * ss + xl], sem)
    for i in (len(planes) - 2, len(planes) - 1):
        c, xl = planes[i]
        stg = (stg0, stg1)[i % 2]
        sem = (sem0, sem1)[i % 2]
        pltpu.make_async_copy(stg,
                              out_hbm.at[bb, c, 8 * ss + xl], sem).wait()


@jax.jit
def kernel(rgb, depth):
    um = jnp.arange(W, dtype=jnp.float32) - 112.0
    mesh = plsc.VectorSubcoreMesh(core_axis_name="c", subcore_axis_name="s")
    call = functools.partial(
        pl.kernel,
        mesh=mesh,
        compiler_params=pltpu.CompilerParams(needs_layout_passes=False),
        out_type=jax.ShapeDtypeStruct((B, 3, VOX, VOX, VOX), jnp.float32),
        scratch_types=[
            pltpu.VMEM((SLAB,), jnp.float32),
            pltpu.VMEM((SLAB,), jnp.float32),
            pltpu.VMEM((SLAB,), jnp.float32),
            pltpu.VMEM((2, RPC, 128), jnp.float32),
            pltpu.VMEM((2, RPC, 96), jnp.float32),
            pltpu.VMEM((2, RPC, 128), jnp.float32),
            pltpu.VMEM((2, RPC, 96), jnp.float32),
            pltpu.VMEM((2, RPC, 128), jnp.float32),
            pltpu.VMEM((2, RPC, 96), jnp.float32),
            pltpu.VMEM((2, RPC, 128), jnp.float32),
            pltpu.VMEM((2, RPC, 96), jnp.float32),
            pltpu.VMEM((W,), jnp.float32),
            pltpu.VMEM((VOX, VOX), jnp.float32),
            pltpu.VMEM((VOX, VOX), jnp.float32),
            pltpu.SemaphoreType.DMA,
            pltpu.SemaphoreType.DMA,
            pltpu.SemaphoreType.DMA,
        ],
    )(_voxel_body)
    return call(rgb, depth, um)
